# baseline (speedup 1.0000x reference)
import functools

import jax
import jax.numpy as jnp
from jax import lax
from jax.experimental import pallas as pl
from jax.experimental.pallas import tpu as pltpu
from jax.experimental.pallas import tpu_sc as plsc

N_NODES = 10000
N_EDGES = 320000
D_IN = 128
D_HID = 128
D_OUT = 40
D_OUTP = 128

NC = 2
NS = 16
NW = NC * NS
CHUNK = 128
N_CHUNKS = 2528
E_PAD = N_CHUNKS * CHUNK
K_SLOW = 55
K_FAST = 103
SLOW_CORE = 1
K_ROWS = K_FAST
ACC_ROWS = 10240
ROWS_PER_TILE = ACC_ROWS // NS

_mesh = plsc.VectorSubcoreMesh(core_axis_name="c", subcore_axis_name="s")


def _fill_buf(buf, rows, width, value):
    def row_body(i, _):
        def col_body(k, __):
            buf[i, pl.ds(k * 16, 16)] = jnp.full((16,), value, jnp.float32)
            return 0
        return lax.fori_loop(0, width // 16, col_body, 0)
    lax.fori_loop(0, rows, row_body, 0)


def _zero_buf(buf, rows, width):
    _fill_buf(buf, rows, width, 0.0)


DEG_W = 128


def _make_deg_kernel():
    DW = DEG_W

    @functools.partial(
        pl.kernel,
        mesh=_mesh,
        out_type=jax.ShapeDtypeStruct((NC * ACC_ROWS, DW), jnp.float32),
        scratch_types=[
            pltpu.VMEM((K_ROWS, CHUNK), jnp.int32),
            pltpu.VMEM((CHUNK, DW), jnp.float32),
            pltpu.VMEM((CHUNK, DW), jnp.float32),
            pltpu.VMEM_SHARED((ACC_ROWS, DW), jnp.float32),
            pltpu.SemaphoreType.DMA,
        ],
    )
    def deg_kernel(dst_hbm, out_hbm, dst_v, ones_v, zbuf_v, acc_sh, sem):
        c = lax.axis_index("c")
        s = lax.axis_index("s")
        wid = s * NC + c

        _zero_buf(zbuf_v, CHUNK, DW)
        _fill_buf(ones_v, CHUNK, DW, 1.0)

        for r in range(ROWS_PER_TILE // CHUNK):
            pltpu.sync_copy(zbuf_v,
                            acc_sh.at[pl.ds(s * ROWS_PER_TILE + r * CHUNK, CHUNK)])
        plsc.subcore_barrier()

        pltpu.sync_copy(dst_hbm.at[wid], dst_v)

        count = jnp.where(c == SLOW_CORE, K_SLOW, K_FAST)

        def body(j, _):
            pltpu.sync_copy(ones_v, acc_sh.at[dst_v.at[j]], add=True)
            return 0
        lax.fori_loop(0, count, body, 0)
        plsc.subcore_barrier()

        off = c * ACC_ROWS + s * ROWS_PER_TILE
        pltpu.sync_copy(acc_sh.at[pl.ds(s * ROWS_PER_TILE, ROWS_PER_TILE)],
                        out_hbm.at[pl.ds(off, ROWS_PER_TILE)])

    return deg_kernel


def _make_agg_kernel(D):
    @functools.partial(
        pl.kernel,
        mesh=_mesh,
        out_type=jax.ShapeDtypeStruct((NC * ACC_ROWS, D), jnp.float32),
        scratch_types=[
            pltpu.VMEM((K_ROWS, CHUNK), jnp.int32),
            pltpu.VMEM((K_ROWS, CHUNK), jnp.int32),
            pltpu.VMEM((CHUNK, D), jnp.float32),
            pltpu.VMEM_SHARED((ACC_ROWS, D), jnp.float32),
            pltpu.SemaphoreType.DMA,
        ],
    )
    def agg_kernel(table_hbm, src_hbm, dst_hbm, out_hbm,
                   src_v, dst_v, buf_v, acc_sh, gsem):
        c = lax.axis_index("c")
        s = lax.axis_index("s")
        wid = s * NC + c

        _zero_buf(buf_v, CHUNK, D)
        for r in range(ROWS_PER_TILE // CHUNK):
            pltpu.sync_copy(buf_v,
                            acc_sh.at[pl.ds(s * ROWS_PER_TILE + r * CHUNK, CHUNK)])
        plsc.subcore_barrier()

        pltpu.sync_copy(src_hbm.at[wid], src_v)
        pltpu.sync_copy(dst_hbm.at[wid], dst_v)

        count = jnp.where(c == SLOW_CORE, K_SLOW, K_FAST)

        def body(j, _):
            pltpu.async_copy(table_hbm.at[src_v.at[j]], buf_v, gsem).wait()
            pltpu.sync_copy(buf_v, acc_sh.at[dst_v.at[j]], add=True)
            return 0
        lax.fori_loop(0, count, body, 0)
        plsc.subcore_barrier()

        off = c * ACC_ROWS + s * ROWS_PER_TILE
        pltpu.sync_copy(acc_sh.at[pl.ds(s * ROWS_PER_TILE, ROWS_PER_TILE)],
                        out_hbm.at[pl.ds(off, ROWS_PER_TILE)])

    return agg_kernel


_deg_kernel = _make_deg_kernel()
_agg128 = _make_agg_kernel(D_HID)


_BN = 1000
_GRID = N_NODES // _BN


def _dinv_block(p0, p1):
    deg = p0[:, 0:1] + p1[:, 0:1] + 1.0
    return lax.rsqrt(deg)


def _tc_a_body(x_ref, w1_ref, p0_ref, p1_ref, xs1_ref):
    dinv = _dinv_block(p0_ref[...], p1_ref[...])
    h = jnp.dot(x_ref[...], w1_ref[...], preferred_element_type=jnp.float32)
    xs1_ref[...] = h * dinv


def _tc_b_body(a0_ref, a1_ref, xs1_ref, p0_ref, p1_ref, b1_ref, w2_ref, xs2_ref):
    dinv = _dinv_block(p0_ref[...], p1_ref[...])
    z = dinv * (a0_ref[...] + a1_ref[...] + xs1_ref[...]) + b1_ref[...]
    r = jnp.maximum(z, 0.0)
    g = jnp.dot(r, w2_ref[...], preferred_element_type=jnp.float32)
    xs2_ref[...] = g * dinv


def _tc_c_body(c0_ref, c1_ref, xs2_ref, p0_ref, p1_ref, b2_ref, out_ref):
    dinv = _dinv_block(p0_ref[...], p1_ref[...])
    o = dinv * (c0_ref[...] + c1_ref[...] + xs2_ref[...]) + b2_ref[...]
    mask = lax.broadcasted_iota(jnp.int32, (1, D_OUTP), 1) < D_OUT
    om = jnp.where(mask, o, jnp.float32(-1e30))
    m = jnp.max(om, axis=1, keepdims=True)
    e = jnp.where(mask, jnp.exp(o - m), 0.0)
    ssum = jnp.sum(e, axis=1, keepdims=True)
    out_ref[...] = o - m - jnp.log(ssum)


def _row_spec(width):
    return pl.BlockSpec((_BN, width), lambda i: (i, 0))


def _full_spec(shape):
    return pl.BlockSpec(shape, lambda i: (0, 0))


def _tc_a(x, W1, p0, p1):
    return pl.pallas_call(
        _tc_a_body,
        grid=(_GRID,),
        in_specs=[_row_spec(D_IN), _full_spec((D_IN, D_HID)),
                  _row_spec(16), _row_spec(16)],
        out_specs=_row_spec(D_HID),
        out_shape=jax.ShapeDtypeStruct((N_NODES, D_HID), jnp.float32),
    )(x, W1, p0, p1)


def _tc_b(a0, a1, xs1, p0, p1, b1r, W2p):
    return pl.pallas_call(
        _tc_b_body,
        grid=(_GRID,),
        in_specs=[_row_spec(D_HID), _row_spec(D_HID), _row_spec(D_HID),
                  _row_spec(16), _row_spec(16),
                  _full_spec((1, D_HID)), _full_spec((D_HID, D_OUTP))],
        out_specs=_row_spec(D_OUTP),
        out_shape=jax.ShapeDtypeStruct((N_NODES, D_OUTP), jnp.float32),
    )(a0, a1, xs1, p0, p1, b1r, W2p)


def _tc_c(c0, c1, xs2, p0, p1, b2r):
    return pl.pallas_call(
        _tc_c_body,
        grid=(_GRID,),
        in_specs=[_row_spec(D_OUTP), _row_spec(D_OUTP), _row_spec(D_OUTP),
                  _row_spec(16), _row_spec(16), _full_spec((1, D_OUTP))],
        out_specs=_row_spec(D_OUTP),
        out_shape=jax.ShapeDtypeStruct((N_NODES, D_OUTP), jnp.float32),
    )(c0, c1, xs2, p0, p1, b2r)


def _build_chunk_idx():
    import numpy as _np
    idx = _np.full((NW, K_ROWS), N_CHUNKS, _np.int32)
    per_pair = N_CHUNKS // NS
    for s in range(NS):
        base = s * per_pair
        w_slow = s * NC + SLOW_CORE
        w_fast = s * NC + (1 - SLOW_CORE)
        idx[w_slow, :K_SLOW] = _np.arange(base, base + K_SLOW)
        idx[w_fast, :K_FAST] = _np.arange(base + K_SLOW, base + per_pair)
    return idx


_CHUNK_IDX = _build_chunk_idx()

def kernel(x, edge_index, W1, b1, W2, b2):
    src = edge_index[0].astype(jnp.int32)
    dst = edge_index[1].astype(jnp.int32)
    pad = E_PAD - N_EDGES
    srcp = jnp.concatenate([src, jnp.zeros((pad, ), jnp.int32),
                            jnp.zeros((CHUNK,), jnp.int32)])
    dstp = jnp.concatenate([dst, jnp.full((pad,), N_NODES, jnp.int32),
                            jnp.full((CHUNK,), N_NODES, jnp.int32)])
    srcp = srcp.reshape(N_CHUNKS + 1, CHUNK)[_CHUNK_IDX]
    dstp = dstp.reshape(N_CHUNKS + 1, CHUNK)[_CHUNK_IDX]

    degp = _deg_kernel(dstp)
    p0 = degp[:N_NODES, :16]
    p1 = degp[ACC_ROWS:ACC_ROWS + N_NODES, :16]

    xs1 = _tc_a(x, W1, p0, p1)

    agg1 = _agg128(xs1, srcp, dstp)
    a0 = agg1[:N_NODES]
    a1 = agg1[ACC_ROWS:ACC_ROWS + N_NODES]

    W2p = jnp.pad(W2, ((0, 0), (0, D_OUTP - D_OUT)))
    b1r = b1.reshape(1, D_HID)
    b2r = jnp.pad(b2, (0, D_OUTP - D_OUT)).reshape(1, D_OUTP)

    xs2 = _tc_b(a0, a1, xs1, p0, p1, b1r, W2p)

    agg2 = _agg128(xs2, srcp, dstp)
    c0 = agg2[:N_NODES]
    c1 = agg2[ACC_ROWS:ACC_ROWS + N_NODES]

    out = _tc_c(c0, c1, xs2, p0, p1, b2r)
    return out[:, :D_OUT]


# speedup vs baseline: 1.1484x; 1.1484x over previous
import functools

import jax
import jax.numpy as jnp
from jax import lax
from jax.experimental import pallas as pl
from jax.experimental.pallas import tpu as pltpu
from jax.experimental.pallas import tpu_sc as plsc

N_NODES = 10000
N_EDGES = 320000
D_IN = 128
D_HID = 128
D_OUT = 40
D_OUTP = 128

NC = 2
NS = 16
NW = NC * NS
CHUNK = 128
K_CHUNKS = 79
EDGES_PER_WORKER = K_CHUNKS * CHUNK
E_PAD = NW * EDGES_PER_WORKER
K_ROWS = K_CHUNKS
ACC_ROWS = 10240
ROWS_PER_TILE = ACC_ROWS // NS

_mesh = plsc.VectorSubcoreMesh(core_axis_name="c", subcore_axis_name="s")


def _fill_buf(buf, rows, width, value):
    def row_body(i, _):
        def col_body(k, __):
            buf[i, pl.ds(k * 16, 16)] = jnp.full((16,), value, jnp.float32)
            return 0
        return lax.fori_loop(0, width // 16, col_body, 0)
    lax.fori_loop(0, rows, row_body, 0)


def _zero_buf(buf, rows, width):
    _fill_buf(buf, rows, width, 0.0)


DEG_W = 128


def _make_deg_kernel():
    DW = DEG_W

    @functools.partial(
        pl.kernel,
        mesh=_mesh,
        out_type=jax.ShapeDtypeStruct((NC * ACC_ROWS, DW), jnp.float32),
        scratch_types=[
            pltpu.VMEM((K_ROWS, CHUNK), jnp.int32),
            pltpu.VMEM((CHUNK, DW), jnp.float32),
            pltpu.VMEM((CHUNK, DW), jnp.float32),
            pltpu.VMEM_SHARED((ACC_ROWS, DW), jnp.float32),
            pltpu.SemaphoreType.DMA,
        ],
    )
    def deg_kernel(dst_hbm, ones_hbm, zrows_hbm, out_hbm,
                   dst_v, ones_v, zbuf_v, acc_sh, sem):
        c = lax.axis_index("c")
        s = lax.axis_index("s")
        wid = s * NC + c

        pltpu.sync_copy(ones_hbm, ones_v)
        pltpu.sync_copy(zrows_hbm, zbuf_v)
        for r in range(ROWS_PER_TILE // CHUNK):
            pltpu.sync_copy(zbuf_v,
                            acc_sh.at[pl.ds(s * ROWS_PER_TILE + r * CHUNK, CHUNK)])
        plsc.subcore_barrier()

        pltpu.sync_copy(dst_hbm.at[wid], dst_v)

        def body(j, _):
            pltpu.sync_copy(ones_v, acc_sh.at[dst_v.at[j]], add=True)
            return 0
        lax.fori_loop(0, K_CHUNKS, body, 0)
        plsc.subcore_barrier()

        off = c * ACC_ROWS + s * ROWS_PER_TILE
        pltpu.sync_copy(acc_sh.at[pl.ds(s * ROWS_PER_TILE, ROWS_PER_TILE)],
                        out_hbm.at[pl.ds(off, ROWS_PER_TILE)])

    return deg_kernel


def _make_agg_kernel(D):
    @functools.partial(
        pl.kernel,
        mesh=_mesh,
        out_type=jax.ShapeDtypeStruct((NC * ACC_ROWS, D), jnp.float32),
        scratch_types=[
            pltpu.VMEM((K_ROWS, CHUNK), jnp.int32),
            pltpu.VMEM((K_ROWS, CHUNK), jnp.int32),
            pltpu.VMEM((CHUNK, D), jnp.float32),
            pltpu.VMEM_SHARED((ACC_ROWS, D), jnp.float32),
            pltpu.SemaphoreType.DMA,
        ],
    )
    def agg_kernel(table_hbm, src_hbm, dst_hbm, zrows_hbm, out_hbm,
                   src_v, dst_v, buf_v, acc_sh, gsem):
        c = lax.axis_index("c")
        s = lax.axis_index("s")
        wid = s * NC + c

        pltpu.sync_copy(zrows_hbm, buf_v)
        for r in range(ROWS_PER_TILE // CHUNK):
            pltpu.sync_copy(buf_v,
                            acc_sh.at[pl.ds(s * ROWS_PER_TILE + r * CHUNK, CHUNK)])
        plsc.subcore_barrier()

        pltpu.sync_copy(src_hbm.at[wid], src_v)
        pltpu.sync_copy(dst_hbm.at[wid], dst_v)

        def body(j, _):
            pltpu.async_copy(table_hbm.at[src_v.at[j]], buf_v, gsem).wait()
            pltpu.sync_copy(buf_v, acc_sh.at[dst_v.at[j]], add=True)
            return 0
        lax.fori_loop(0, K_CHUNKS, body, 0)
        plsc.subcore_barrier()

        off = c * ACC_ROWS + s * ROWS_PER_TILE
        pltpu.sync_copy(acc_sh.at[pl.ds(s * ROWS_PER_TILE, ROWS_PER_TILE)],
                        out_hbm.at[pl.ds(off, ROWS_PER_TILE)])

    return agg_kernel


_deg_kernel = _make_deg_kernel()
_agg128 = _make_agg_kernel(D_HID)


_BN = 1000
_GRID = N_NODES // _BN


def _dinv_block(p0, p1):
    deg = p0[:, 0:1] + p1[:, 0:1] + 1.0
    return lax.rsqrt(deg)


def _tc_a_body(x_ref, w1_ref, p0_ref, p1_ref, xs1_ref):
    dinv = _dinv_block(p0_ref[...], p1_ref[...])
    h = jnp.dot(x_ref[...], w1_ref[...], preferred_element_type=jnp.float32)
    xs1_ref[...] = h * dinv


def _tc_b_body(a0_ref, a1_ref, xs1_ref, p0_ref, p1_ref, b1_ref, w2_ref, xs2_ref):
    dinv = _dinv_block(p0_ref[...], p1_ref[...])
    z = dinv * (a0_ref[...] + a1_ref[...] + xs1_ref[...]) + b1_ref[...]
    r = jnp.maximum(z, 0.0)
    g = jnp.dot(r, w2_ref[...], preferred_element_type=jnp.float32)
    xs2_ref[...] = g * dinv


def _tc_c_body(c0_ref, c1_ref, xs2_ref, p0_ref, p1_ref, b2_ref, out_ref):
    dinv = _dinv_block(p0_ref[...], p1_ref[...])
    o = dinv * (c0_ref[...] + c1_ref[...] + xs2_ref[...]) + b2_ref[...]
    mask = lax.broadcasted_iota(jnp.int32, (1, D_OUTP), 1) < D_OUT
    om = jnp.where(mask, o, jnp.float32(-1e30))
    m = jnp.max(om, axis=1, keepdims=True)
    e = jnp.where(mask, jnp.exp(o - m), 0.0)
    ssum = jnp.sum(e, axis=1, keepdims=True)
    out_ref[...] = o - m - jnp.log(ssum)


def _row_spec(width):
    return pl.BlockSpec((_BN, width), lambda i: (i, 0))


def _full_spec(shape):
    return pl.BlockSpec(shape, lambda i: (0, 0))


def _tc_a(x, W1, p0, p1):
    return pl.pallas_call(
        _tc_a_body,
        grid=(_GRID,),
        in_specs=[_row_spec(D_IN), _full_spec((D_IN, D_HID)),
                  _row_spec(16), _row_spec(16)],
        out_specs=_row_spec(D_HID),
        out_shape=jax.ShapeDtypeStruct((N_NODES, D_HID), jnp.float32),
    )(x, W1, p0, p1)


def _tc_b(a0, a1, xs1, p0, p1, b1r, W2p):
    return pl.pallas_call(
        _tc_b_body,
        grid=(_GRID,),
        in_specs=[_row_spec(D_HID), _row_spec(D_HID), _row_spec(D_HID),
                  _row_spec(16), _row_spec(16),
                  _full_spec((1, D_HID)), _full_spec((D_HID, D_OUTP))],
        out_specs=_row_spec(D_OUTP),
        out_shape=jax.ShapeDtypeStruct((N_NODES, D_OUTP), jnp.float32),
    )(a0, a1, xs1, p0, p1, b1r, W2p)


def _tc_c(c0, c1, xs2, p0, p1, b2r):
    return pl.pallas_call(
        _tc_c_body,
        grid=(_GRID,),
        in_specs=[_row_spec(D_OUTP), _row_spec(D_OUTP), _row_spec(D_OUTP),
                  _row_spec(16), _row_spec(16), _full_spec((1, D_OUTP))],
        out_specs=_row_spec(D_OUTP),
        out_shape=jax.ShapeDtypeStruct((N_NODES, D_OUTP), jnp.float32),
    )(c0, c1, xs2, p0, p1, b2r)


def kernel(x, edge_index, W1, b1, W2, b2):
    src = edge_index[0].astype(jnp.int32)
    dst = edge_index[1].astype(jnp.int32)
    pad = E_PAD - N_EDGES
    srcp = jnp.concatenate([src, jnp.zeros((pad,), jnp.int32)])
    dstp = jnp.concatenate([dst, jnp.full((pad,), N_NODES, jnp.int32)])
    srcp = srcp.reshape(NW, K_CHUNKS, CHUNK)
    dstp = dstp.reshape(NW, K_CHUNKS, CHUNK)

    ones_c = jnp.ones((CHUNK, DEG_W), jnp.float32)
    zrows = jnp.zeros((CHUNK, DEG_W), jnp.float32)
    degp = _deg_kernel(dstp, ones_c, zrows)
    p0 = degp[:N_NODES, :16]
    p1 = degp[ACC_ROWS:ACC_ROWS + N_NODES, :16]

    xs1 = _tc_a(x, W1, p0, p1)

    agg1 = _agg128(xs1, srcp, dstp, zrows)
    a0 = agg1[:N_NODES]
    a1 = agg1[ACC_ROWS:ACC_ROWS + N_NODES]

    W2p = jnp.pad(W2, ((0, 0), (0, D_OUTP - D_OUT)))
    b1r = b1.reshape(1, D_HID)
    b2r = jnp.pad(b2, (0, D_OUTP - D_OUT)).reshape(1, D_OUTP)

    xs2 = _tc_b(a0, a1, xs1, p0, p1, b1r, W2p)

    agg2 = _agg128(xs2, srcp, dstp, zrows)
    c0 = agg2[:N_NODES]
    c1 = agg2[ACC_ROWS:ACC_ROWS + N_NODES]

    out = _tc_c(c0, c1, xs2, p0, p1, b2r)
    return out[:, :D_OUT]


# speedup vs baseline: 1.1739x; 1.0223x over previous
import functools

import jax
import jax.numpy as jnp
from jax import lax
from jax.experimental import pallas as pl
from jax.experimental.pallas import tpu as pltpu
from jax.experimental.pallas import tpu_sc as plsc

N_NODES = 10000
N_EDGES = 320000
D_IN = 128
D_HID = 128
D_OUT = 40
D_OUTP = 128

NC = 2
NS = 16
NW = NC * NS
CHUNK = 128
K_CHUNKS = 79
EDGES_PER_WORKER = K_CHUNKS * CHUNK
E_PAD = NW * EDGES_PER_WORKER
K_ROWS = K_CHUNKS
ACC_ROWS = 10240
ROWS_PER_TILE = ACC_ROWS // NS

_mesh = plsc.VectorSubcoreMesh(core_axis_name="c", subcore_axis_name="s")


def _fill_buf(buf, rows, width, value):
    def row_body(i, _):
        def col_body(k, __):
            buf[i, pl.ds(k * 16, 16)] = jnp.full((16,), value, jnp.float32)
            return 0
        return lax.fori_loop(0, width // 16, col_body, 0)
    lax.fori_loop(0, rows, row_body, 0)


def _zero_buf(buf, rows, width):
    _fill_buf(buf, rows, width, 0.0)


DEG_W = 128


def _make_deg_kernel():
    DW = DEG_W

    @functools.partial(
        pl.kernel,
        mesh=_mesh,
        out_type=jax.ShapeDtypeStruct((NC * ACC_ROWS, DW), jnp.float32),
        scratch_types=[
            pltpu.VMEM((K_ROWS, CHUNK), jnp.int32),
            pltpu.VMEM((CHUNK, DW), jnp.float32),
            pltpu.VMEM((CHUNK, DW), jnp.float32),
            pltpu.VMEM_SHARED((ACC_ROWS, DW), jnp.float32),
            pltpu.SemaphoreType.DMA,
        ],
    )
    def deg_kernel(dst_hbm, out_hbm, dst_v, ones_v, zbuf_v, acc_sh, sem):
        c = lax.axis_index("c")
        s = lax.axis_index("s")
        wid = s * NC + c

        _zero_buf(zbuf_v, CHUNK, DW)
        _fill_buf(ones_v, CHUNK, DW, 1.0)

        for r in range(ROWS_PER_TILE // CHUNK):
            pltpu.sync_copy(zbuf_v,
                            acc_sh.at[pl.ds(s * ROWS_PER_TILE + r * CHUNK, CHUNK)])
        plsc.subcore_barrier()

        pltpu.sync_copy(dst_hbm.at[wid], dst_v)

        def body(j, _):
            pltpu.sync_copy(ones_v, acc_sh.at[dst_v.at[j]], add=True)
            return 0
        lax.fori_loop(0, K_CHUNKS, body, 0)
        plsc.subcore_barrier()

        off = c * ACC_ROWS + s * ROWS_PER_TILE
        pltpu.sync_copy(acc_sh.at[pl.ds(s * ROWS_PER_TILE, ROWS_PER_TILE)],
                        out_hbm.at[pl.ds(off, ROWS_PER_TILE)])

    return deg_kernel


def _make_agg_kernel(D):
    @functools.partial(
        pl.kernel,
        mesh=_mesh,
        out_type=jax.ShapeDtypeStruct((NC * ACC_ROWS, D), jnp.float32),
        scratch_types=[
            pltpu.VMEM((K_ROWS, CHUNK), jnp.int32),
            pltpu.VMEM((K_ROWS, CHUNK), jnp.int32),
            pltpu.VMEM((CHUNK, D), jnp.float32),
            pltpu.VMEM_SHARED((ACC_ROWS, D), jnp.float32),
            pltpu.SemaphoreType.DMA,
        ],
    )
    def agg_kernel(table_hbm, src_hbm, dst_hbm, out_hbm,
                   src_v, dst_v, buf_v, acc_sh, gsem):
        c = lax.axis_index("c")
        s = lax.axis_index("s")
        wid = s * NC + c

        _zero_buf(buf_v, CHUNK, D)
        for r in range(ROWS_PER_TILE // CHUNK):
            pltpu.sync_copy(buf_v,
                            acc_sh.at[pl.ds(s * ROWS_PER_TILE + r * CHUNK, CHUNK)])
        plsc.subcore_barrier()

        pltpu.sync_copy(src_hbm.at[wid], src_v)
        pltpu.sync_copy(dst_hbm.at[wid], dst_v)

        def body(j, _):
            pltpu.async_copy(table_hbm.at[src_v.at[j]], buf_v, gsem).wait()
            pltpu.sync_copy(buf_v, acc_sh.at[dst_v.at[j]], add=True)
            return 0
        lax.fori_loop(0, K_CHUNKS, body, 0)
        plsc.subcore_barrier()

        off = c * ACC_ROWS + s * ROWS_PER_TILE
        pltpu.sync_copy(acc_sh.at[pl.ds(s * ROWS_PER_TILE, ROWS_PER_TILE)],
                        out_hbm.at[pl.ds(off, ROWS_PER_TILE)])

    return agg_kernel


_deg_kernel = _make_deg_kernel()
_agg128 = _make_agg_kernel(D_HID)


_BN = 1000
_GRID = N_NODES // _BN


def _dinv_block(p0, p1):
    deg = p0[:, 0:1] + p1[:, 0:1] + 1.0
    return lax.rsqrt(deg)


def _tc_a1_body(x_ref, w1_ref, h_ref):
    h_ref[...] = jnp.dot(x_ref[...], w1_ref[...],
                         preferred_element_type=jnp.float32)


def _tc_a2_body(h_ref, p0_ref, p1_ref, xs1_ref):
    dinv = _dinv_block(p0_ref[...], p1_ref[...])
    xs1_ref[...] = h_ref[...] * dinv


def _tc_b_body(a0_ref, a1_ref, xs1_ref, p0_ref, p1_ref, b1_ref, w2_ref, xs2_ref):
    dinv = _dinv_block(p0_ref[...], p1_ref[...])
    z = dinv * (a0_ref[...] + a1_ref[...] + xs1_ref[...]) + b1_ref[...]
    r = jnp.maximum(z, 0.0)
    g = jnp.dot(r, w2_ref[...], preferred_element_type=jnp.float32)
    xs2_ref[...] = g * dinv


def _tc_c_body(c0_ref, c1_ref, xs2_ref, p0_ref, p1_ref, b2_ref, out_ref):
    dinv = _dinv_block(p0_ref[...], p1_ref[...])
    o = dinv * (c0_ref[...] + c1_ref[...] + xs2_ref[...]) + b2_ref[...]
    mask = lax.broadcasted_iota(jnp.int32, (1, D_OUTP), 1) < D_OUT
    om = jnp.where(mask, o, jnp.float32(-1e30))
    m = jnp.max(om, axis=1, keepdims=True)
    e = jnp.where(mask, jnp.exp(o - m), 0.0)
    ssum = jnp.sum(e, axis=1, keepdims=True)
    out_ref[...] = o - m - jnp.log(ssum)


def _row_spec(width):
    return pl.BlockSpec((_BN, width), lambda i: (i, 0))


def _full_spec(shape):
    return pl.BlockSpec(shape, lambda i: (0, 0))


def _tc_a1(x, W1):
    return pl.pallas_call(
        _tc_a1_body,
        grid=(_GRID,),
        in_specs=[_row_spec(D_IN), _full_spec((D_IN, D_HID))],
        out_specs=_row_spec(D_HID),
        out_shape=jax.ShapeDtypeStruct((N_NODES, D_HID), jnp.float32),
    )(x, W1)


def _tc_a2(h, p0, p1):
    return pl.pallas_call(
        _tc_a2_body,
        grid=(_GRID,),
        in_specs=[_row_spec(D_HID), _row_spec(16), _row_spec(16)],
        out_specs=_row_spec(D_HID),
        out_shape=jax.ShapeDtypeStruct((N_NODES, D_HID), jnp.float32),
    )(h, p0, p1)


def _tc_b(a0, a1, xs1, p0, p1, b1r, W2p):
    return pl.pallas_call(
        _tc_b_body,
        grid=(_GRID,),
        in_specs=[_row_spec(D_HID), _row_spec(D_HID), _row_spec(D_HID),
                  _row_spec(16), _row_spec(16),
                  _full_spec((1, D_HID)), _full_spec((D_HID, D_OUTP))],
        out_specs=_row_spec(D_OUTP),
        out_shape=jax.ShapeDtypeStruct((N_NODES, D_OUTP), jnp.float32),
    )(a0, a1, xs1, p0, p1, b1r, W2p)


def _tc_c(c0, c1, xs2, p0, p1, b2r):
    return pl.pallas_call(
        _tc_c_body,
        grid=(_GRID,),
        in_specs=[_row_spec(D_OUTP), _row_spec(D_OUTP), _row_spec(D_OUTP),
                  _row_spec(16), _row_spec(16), _full_spec((1, D_OUTP))],
        out_specs=_row_spec(D_OUTP),
        out_shape=jax.ShapeDtypeStruct((N_NODES, D_OUTP), jnp.float32),
    )(c0, c1, xs2, p0, p1, b2r)


def kernel(x, edge_index, W1, b1, W2, b2):
    src = edge_index[0].astype(jnp.int32)
    dst = edge_index[1].astype(jnp.int32)
    pad = E_PAD - N_EDGES
    srcp = jnp.concatenate([src, jnp.zeros((pad,), jnp.int32)])
    dstp = jnp.concatenate([dst, jnp.full((pad,), N_NODES, jnp.int32)])
    srcp = srcp.reshape(NW, K_CHUNKS, CHUNK)
    dstp = dstp.reshape(NW, K_CHUNKS, CHUNK)

    degp = _deg_kernel(dstp)
    p0 = degp[:N_NODES, :16]
    p1 = degp[ACC_ROWS:ACC_ROWS + N_NODES, :16]

    h1 = _tc_a1(x, W1)
    xs1 = _tc_a2(h1, p0, p1)

    agg1 = _agg128(xs1, srcp, dstp)
    a0 = agg1[:N_NODES]
    a1 = agg1[ACC_ROWS:ACC_ROWS + N_NODES]

    W2p = jnp.pad(W2, ((0, 0), (0, D_OUTP - D_OUT)))
    b1r = b1.reshape(1, D_HID)
    b2r = jnp.pad(b2, (0, D_OUTP - D_OUT)).reshape(1, D_OUTP)

    xs2 = _tc_b(a0, a1, xs1, p0, p1, b1r, W2p)

    agg2 = _agg128(xs2, srcp, dstp)
    c0 = agg2[:N_NODES]
    c1 = agg2[ACC_ROWS:ACC_ROWS + N_NODES]

    out = _tc_c(c0, c1, xs2, p0, p1, b2r)
    return out[:, :D_OUT]


# speedup vs baseline: 1.2283x; 1.0463x over previous
import functools

import jax
import jax.numpy as jnp
from jax import lax
from jax.experimental import pallas as pl
from jax.experimental.pallas import tpu as pltpu
from jax.experimental.pallas import tpu_sc as plsc

N_NODES = 10000
N_EDGES = 320000
D_IN = 128
D_HID = 128
D_OUT = 40
D_OUTP = 128

NC = 2
NS = 16
NW = NC * NS
CHUNK = 128
K_CHUNKS = 79
EDGES_PER_WORKER = K_CHUNKS * CHUNK
E_PAD = NW * EDGES_PER_WORKER
K_ROWS = K_CHUNKS
ACC_ROWS = 10240
ROWS_PER_TILE = ACC_ROWS // NS

_mesh = plsc.VectorSubcoreMesh(core_axis_name="c", subcore_axis_name="s")


def _fill_buf(buf, rows, width, value):
    def row_body(i, _):
        def col_body(k, __):
            buf[i, pl.ds(k * 16, 16)] = jnp.full((16,), value, jnp.float32)
            return 0
        return lax.fori_loop(0, width // 16, col_body, 0)
    lax.fori_loop(0, rows, row_body, 0)


def _zero_buf(buf, rows, width):
    _fill_buf(buf, rows, width, 0.0)


DEG_W = 128


def _make_deg_kernel():
    DW = DEG_W

    @functools.partial(
        pl.kernel,
        mesh=_mesh,
        out_type=jax.ShapeDtypeStruct((NC * ACC_ROWS, DW), jnp.float32),
        scratch_types=[
            pltpu.VMEM((K_ROWS, CHUNK), jnp.int32),
            pltpu.VMEM((CHUNK, DW), jnp.float32),
            pltpu.VMEM((CHUNK, DW), jnp.float32),
            pltpu.VMEM_SHARED((ACC_ROWS, DW), jnp.float32),
            pltpu.SemaphoreType.DMA,
        ],
    )
    def deg_kernel(dst_hbm, out_hbm, dst_v, ones_v, zbuf_v, acc_sh, sem):
        c = lax.axis_index("c")
        s = lax.axis_index("s")
        wid = s * NC + c

        _zero_buf(zbuf_v, CHUNK, DW)
        _fill_buf(ones_v, CHUNK, DW, 1.0)

        for r in range(ROWS_PER_TILE // CHUNK):
            pltpu.sync_copy(zbuf_v,
                            acc_sh.at[pl.ds(s * ROWS_PER_TILE + r * CHUNK, CHUNK)])
        plsc.subcore_barrier()

        pltpu.sync_copy(dst_hbm.at[wid], dst_v)

        def body(j, _):
            pltpu.sync_copy(ones_v, acc_sh.at[dst_v.at[j]], add=True)
            return 0
        lax.fori_loop(0, K_CHUNKS, body, 0)
        plsc.subcore_barrier()

        off = c * ACC_ROWS + s * ROWS_PER_TILE
        pltpu.sync_copy(acc_sh.at[pl.ds(s * ROWS_PER_TILE, ROWS_PER_TILE)],
                        out_hbm.at[pl.ds(off, ROWS_PER_TILE)])

    return deg_kernel


def _make_agg_kernel(D):
    @functools.partial(
        pl.kernel,
        mesh=_mesh,
        out_type=jax.ShapeDtypeStruct((NC * ACC_ROWS, D), jnp.float32),
        scratch_types=[
            pltpu.VMEM((K_ROWS, CHUNK), jnp.int32),
            pltpu.VMEM((K_ROWS, CHUNK), jnp.int32),
            pltpu.VMEM((CHUNK, D), jnp.float32),
            pltpu.VMEM_SHARED((ACC_ROWS, D), jnp.float32),
            pltpu.SemaphoreType.DMA,
        ],
    )
    def agg_kernel(table_hbm, src_hbm, dst_hbm, out_hbm,
                   src_v, dst_v, buf_v, acc_sh, gsem):
        c = lax.axis_index("c")
        s = lax.axis_index("s")
        wid = s * NC + c

        _zero_buf(buf_v, CHUNK, D)
        for r in range(ROWS_PER_TILE // CHUNK):
            pltpu.sync_copy(buf_v,
                            acc_sh.at[pl.ds(s * ROWS_PER_TILE + r * CHUNK, CHUNK)])
        plsc.subcore_barrier()

        pltpu.sync_copy(src_hbm.at[wid], src_v)
        pltpu.sync_copy(dst_hbm.at[wid], dst_v)

        def body(j, _):
            pltpu.async_copy(table_hbm.at[src_v.at[j]], buf_v, gsem).wait()
            pltpu.sync_copy(buf_v, acc_sh.at[dst_v.at[j]], add=True)
            return 0
        lax.fori_loop(0, K_CHUNKS, body, 0)
        plsc.subcore_barrier()

        off = c * ACC_ROWS + s * ROWS_PER_TILE
        pltpu.sync_copy(acc_sh.at[pl.ds(s * ROWS_PER_TILE, ROWS_PER_TILE)],
                        out_hbm.at[pl.ds(off, ROWS_PER_TILE)])

    return agg_kernel


_deg_kernel = _make_deg_kernel()
_agg128 = _make_agg_kernel(D_HID)


_BN = 1000
_GRID = N_NODES // _BN


def _dinv_block(p0, p1):
    deg = p0[:, 0:1] + p1[:, 0:1] + 1.0
    return lax.rsqrt(deg)


def _tc_a_body(x_ref, w1_ref, p0_ref, p1_ref, xs1_ref):
    dinv = _dinv_block(p0_ref[...], p1_ref[...])
    h = jnp.dot(x_ref[...], w1_ref[...], preferred_element_type=jnp.float32)
    xs1_ref[...] = h * dinv


def _tc_b_body(a0_ref, a1_ref, xs1_ref, p0_ref, p1_ref, b1_ref, w2_ref, xs2_ref):
    dinv = _dinv_block(p0_ref[...], p1_ref[...])
    z = dinv * (a0_ref[...] + a1_ref[...] + xs1_ref[...]) + b1_ref[...]
    r = jnp.maximum(z, 0.0)
    g = jnp.dot(r, w2_ref[...], preferred_element_type=jnp.float32)
    xs2_ref[...] = g * dinv


def _tc_c_body(c0_ref, c1_ref, xs2_ref, p0_ref, p1_ref, b2_ref, out_ref):
    dinv = _dinv_block(p0_ref[...], p1_ref[...])
    o = dinv * (c0_ref[...] + c1_ref[...] + xs2_ref[...]) + b2_ref[...]
    mask = lax.broadcasted_iota(jnp.int32, (1, D_OUTP), 1) < D_OUT
    om = jnp.where(mask, o, jnp.float32(-1e30))
    m = jnp.max(om, axis=1, keepdims=True)
    e = jnp.where(mask, jnp.exp(o - m), 0.0)
    ssum = jnp.sum(e, axis=1, keepdims=True)
    out_ref[...] = o - m - jnp.log(ssum)


def _row_spec(width):
    return pl.BlockSpec((_BN, width), lambda i: (i, 0))


def _full_spec(shape):
    return pl.BlockSpec(shape, lambda i: (0, 0))


def _tc_a(x, W1, p0, p1):
    return pl.pallas_call(
        _tc_a_body,
        grid=(_GRID,),
        in_specs=[_row_spec(D_IN), _full_spec((D_IN, D_HID)),
                  _row_spec(16), _row_spec(16)],
        out_specs=_row_spec(D_HID),
        out_shape=jax.ShapeDtypeStruct((N_NODES, D_HID), jnp.float32),
    )(x, W1, p0, p1)


def _tc_b(a0, a1, xs1, p0, p1, b1r, W2p):
    return pl.pallas_call(
        _tc_b_body,
        grid=(_GRID,),
        in_specs=[_row_spec(D_HID), _row_spec(D_HID), _row_spec(D_HID),
                  _row_spec(16), _row_spec(16),
                  _full_spec((1, D_HID)), _full_spec((D_HID, D_OUTP))],
        out_specs=_row_spec(D_OUTP),
        out_shape=jax.ShapeDtypeStruct((N_NODES, D_OUTP), jnp.float32),
    )(a0, a1, xs1, p0, p1, b1r, W2p)


def _tc_c(c0, c1, xs2, p0, p1, b2r):
    return pl.pallas_call(
        _tc_c_body,
        grid=(_GRID,),
        in_specs=[_row_spec(D_OUTP), _row_spec(D_OUTP), _row_spec(D_OUTP),
                  _row_spec(16), _row_spec(16), _full_spec((1, D_OUTP))],
        out_specs=_row_spec(D_OUTP),
        out_shape=jax.ShapeDtypeStruct((N_NODES, D_OUTP), jnp.float32),
    )(c0, c1, xs2, p0, p1, b2r)


def kernel(x, edge_index, W1, b1, W2, b2):
    src = edge_index[0].astype(jnp.int32)
    dst = edge_index[1].astype(jnp.int32)
    pad = E_PAD - N_EDGES
    srcp = jnp.concatenate([src, jnp.zeros((pad,), jnp.int32)])
    dstp = jnp.concatenate([dst, jnp.full((pad,), N_NODES, jnp.int32)])
    srcp = srcp.reshape(NW, K_CHUNKS, CHUNK)
    dstp = dstp.reshape(NW, K_CHUNKS, CHUNK)

    degp = _deg_kernel(dstp)
    p0 = degp[:N_NODES, :16]
    p1 = degp[ACC_ROWS:ACC_ROWS + N_NODES, :16]

    xs1 = _tc_a(x, W1, p0, p1)

    agg1 = _agg128(xs1, srcp, dstp)
    a0 = agg1[:N_NODES]
    a1 = agg1[ACC_ROWS:ACC_ROWS + N_NODES]

    W2p = jnp.pad(W2, ((0, 0), (0, D_OUTP - D_OUT)))
    b1r = b1.reshape(1, D_HID)
    b2r = jnp.pad(b2, (0, D_OUTP - D_OUT)).reshape(1, D_OUTP)

    xs2 = _tc_b(a0, a1, xs1, p0, p1, b1r, W2p)

    agg2 = _agg128(xs2, srcp, dstp)
    c0 = agg2[:N_NODES]
    c1 = agg2[ACC_ROWS:ACC_ROWS + N_NODES]

    out = _tc_c(c0, c1, xs2, p0, p1, b2r)
    return out[:, :D_OUT]
